# Initial kernel scaffold; baseline (speedup 1.0000x reference)
import jax
import jax.numpy as jnp
from jax.experimental import pallas as pl


def kernel(x, edge_index, W1, b1, W2, b2, W3, b3):
    raise NotImplementedError("write your pallas kernel here")


# speedup vs baseline: 10.4206x; 10.4206x over previous
import jax
import jax.numpy as jnp
from jax import lax
from jax.experimental import pallas as pl
from jax.experimental.pallas import tpu as pltpu
from jax.experimental.pallas import tpu_sc as plsc

NC, NS = 2, 16
NW = NC * NS
N = 10000
D = 128
E = 320000
CH = 128
K = -(-E // (NW * CH))
E_PAD = NW * K * CH
ROWS_PER_TILE = 640
N_PAD = NS * ROWS_PER_TILE
ZR = 64

_mesh = plsc.VectorSubcoreMesh(core_axis_name="c", subcore_axis_name="s",
                               num_cores=NC, num_subcores=NS)


def _sc_degree(dstp, ones_h, zeros_h):
    def body(dst_hbm, ones_hbm, z_hbm, out_hbm, acc, idx_v, ones_v, z_v):
        c = lax.axis_index("c")
        s = lax.axis_index("s")
        wid = c * NS + s
        pltpu.sync_copy(z_hbm, z_v)
        pltpu.sync_copy(z_v, acc.at[pl.ds(s * ROWS_PER_TILE, ROWS_PER_TILE)])
        pltpu.sync_copy(ones_hbm, ones_v)
        pltpu.sync_copy(dst_hbm.at[wid], idx_v)
        plsc.subcore_barrier()

        def step(j, carry):
            pltpu.sync_copy(ones_v, acc.at[idx_v.at[j]], add=True)
            return carry

        lax.fori_loop(0, K, step, 0)
        plsc.subcore_barrier()
        base = c * N_PAD + s * ROWS_PER_TILE
        pltpu.sync_copy(acc.at[pl.ds(s * ROWS_PER_TILE, ROWS_PER_TILE)],
                        out_hbm.at[pl.ds(base, ROWS_PER_TILE)])

    return pl.kernel(
        body,
        out_type=jax.ShapeDtypeStruct((NC * N_PAD,), jnp.float32),
        mesh=_mesh,
        scratch_types=[
            pltpu.VMEM_SHARED((N_PAD,), jnp.float32),
            pltpu.VMEM((K, CH), jnp.int32),
            pltpu.VMEM((CH,), jnp.float32),
            pltpu.VMEM((ROWS_PER_TILE,), jnp.float32),
        ],
    )(dstp, ones_h, zeros_h)


def _sc_edges(g, srcp, dstp, zrows_h):
    def body(g_hbm, src_hbm, dst_hbm, z_hbm, out_hbm,
             acc, src_v, dst_v, rows, zbuf, sem):
        c = lax.axis_index("c")
        s = lax.axis_index("s")
        wid = c * NS + s
        pltpu.sync_copy(z_hbm, zbuf)

        def zstep(k, carry):
            pltpu.sync_copy(zbuf, acc.at[pl.ds(s * ROWS_PER_TILE + k * ZR, ZR)])
            return carry

        lax.fori_loop(0, ROWS_PER_TILE // ZR, zstep, 0)
        pltpu.sync_copy(src_hbm.at[wid], src_v)
        pltpu.sync_copy(dst_hbm.at[wid], dst_v)
        plsc.subcore_barrier()

        def step(j, carry):
            pltpu.async_copy(g_hbm.at[src_v.at[j]], rows, sem).wait()
            pltpu.sync_copy(rows, acc.at[dst_v.at[j]], add=True)
            return carry

        lax.fori_loop(0, K, step, 0)
        plsc.subcore_barrier()
        base = c * N_PAD + s * ROWS_PER_TILE
        pltpu.sync_copy(acc.at[pl.ds(s * ROWS_PER_TILE, ROWS_PER_TILE)],
                        out_hbm.at[pl.ds(base, ROWS_PER_TILE)])

    return pl.kernel(
        body,
        out_type=jax.ShapeDtypeStruct((NC * N_PAD, D), jnp.float32),
        mesh=_mesh,
        scratch_types=[
            pltpu.VMEM_SHARED((N_PAD, D), jnp.float32),
            pltpu.VMEM((K, CH), jnp.int32),
            pltpu.VMEM((K, CH), jnp.int32),
            pltpu.VMEM((CH, D), jnp.float32),
            pltpu.VMEM((ZR, D), jnp.float32),
            pltpu.SemaphoreType.DMA,
        ],
    )(g, srcp, dstp, zrows_h)


def _tc_dinv(degp):
    def body(deg_ref, o_ref):
        o_ref[:] = lax.rsqrt(deg_ref[0, :] + deg_ref[1, :] + 1.0)

    return pl.pallas_call(
        body, out_shape=jax.ShapeDtypeStruct((N_PAD,), jnp.float32))(degp)


BR = 2000


def _tc_mm_scale(x, W, dinv2):
    def body(x_ref, w_ref, di_ref, o_ref):
        o_ref[:] = jnp.dot(x_ref[:], w_ref[:],
                           preferred_element_type=jnp.float32) * di_ref[:]

    return pl.pallas_call(
        body,
        grid=(N // BR,),
        in_specs=[pl.BlockSpec((BR, D), lambda i: (i, 0)),
                  pl.BlockSpec((D, D), lambda i: (0, 0)),
                  pl.BlockSpec((BR, 1), lambda i: (i, 0))],
        out_specs=pl.BlockSpec((BR, D), lambda i: (i, 0)),
        out_shape=jax.ShapeDtypeStruct((N, D), jnp.float32),
    )(x, W, dinv2)


def _tc_combine_mm(s0, s1, g, dinv2, b, W):
    def body(s0_ref, s1_ref, g_ref, di_ref, b_ref, w_ref, o_ref):
        h = di_ref[:] * (s0_ref[:] + s1_ref[:] + g_ref[:]) + b_ref[:]
        h = jnp.maximum(h, 0.0)
        o_ref[:] = jnp.dot(h, w_ref[:],
                           preferred_element_type=jnp.float32) * di_ref[:]

    return pl.pallas_call(
        body,
        grid=(N // BR,),
        in_specs=[pl.BlockSpec((BR, D), lambda i: (i, 0)),
                  pl.BlockSpec((BR, D), lambda i: (i, 0)),
                  pl.BlockSpec((BR, D), lambda i: (i, 0)),
                  pl.BlockSpec((BR, 1), lambda i: (i, 0)),
                  pl.BlockSpec((1, D), lambda i: (0, 0)),
                  pl.BlockSpec((D, D), lambda i: (0, 0))],
        out_specs=pl.BlockSpec((BR, D), lambda i: (i, 0)),
        out_shape=jax.ShapeDtypeStruct((N, D), jnp.float32),
    )(s0, s1, g, dinv2, b, W)


def _tc_final(s0, s1, g, dinv2, b):
    def body(s0_ref, s1_ref, g_ref, di_ref, b_ref, o_ref):
        h = di_ref[:] * (s0_ref[:] + s1_ref[:] + g_ref[:]) + b_ref[:]
        o_ref[:] = jnp.maximum(h, 0.0)

    return pl.pallas_call(
        body,
        grid=(N // BR,),
        in_specs=[pl.BlockSpec((BR, D), lambda i: (i, 0)),
                  pl.BlockSpec((BR, D), lambda i: (i, 0)),
                  pl.BlockSpec((BR, D), lambda i: (i, 0)),
                  pl.BlockSpec((BR, 1), lambda i: (i, 0)),
                  pl.BlockSpec((1, D), lambda i: (0, 0))],
        out_specs=pl.BlockSpec((BR, D), lambda i: (i, 0)),
        out_shape=jax.ShapeDtypeStruct((N, D), jnp.float32),
    )(s0, s1, g, dinv2, b)


def kernel(x, edge_index, W1, b1, W2, b2, W3, b3):
    src = edge_index[0].astype(jnp.int32)
    dst = edge_index[1].astype(jnp.int32)
    pad = E_PAD - E
    srcp = jnp.concatenate([src, jnp.zeros((pad,), jnp.int32)]).reshape(NW, K, CH)
    dstp = jnp.concatenate([dst, jnp.full((pad,), N, jnp.int32)]).reshape(NW, K, CH)
    ones_h = jnp.ones((CH,), jnp.float32)
    z1 = jnp.zeros((ROWS_PER_TILE,), jnp.float32)
    z2 = jnp.zeros((ZR, D), jnp.float32)

    degp = _sc_degree(dstp, ones_h, z1).reshape(NC, N_PAD)
    dinv2 = _tc_dinv(degp)[:N].reshape(N, 1)
    b1r, b2r, b3r = b1.reshape(1, D), b2.reshape(1, D), b3.reshape(1, D)

    g = _tc_mm_scale(x, W1, dinv2)
    s = _sc_edges(g, srcp, dstp, z2).reshape(NC, N_PAD, D)
    g = _tc_combine_mm(s[0, :N], s[1, :N], g, dinv2, b1r, W2)
    s = _sc_edges(g, srcp, dstp, z2).reshape(NC, N_PAD, D)
    g = _tc_combine_mm(s[0, :N], s[1, :N], g, dinv2, b2r, W3)
    s = _sc_edges(g, srcp, dstp, z2).reshape(NC, N_PAD, D)
    return _tc_final(s[0, :N], s[1, :N], g, dinv2, b3r)
